# baseline (speedup 1.0000x reference)
import functools

import jax
import jax.numpy as jnp
from jax import lax
from jax.experimental import pallas as pl
from jax.experimental.pallas import tpu as pltpu
from jax.experimental.pallas import tpu_sc as plsc

BATCH = 16384
FIELD = 26
LANES = 16
NUM_CORES = 2
NUM_SUBCORES = 16
NW = NUM_CORES * NUM_SUBCORES
BPW = BATCH // NW
IPW = BPW * FIELD
FEATS = 1000000


def _lr_body(idx_hbm, w_hbm, bias_hbm, out_hbm, idx_v, rows_v, out_v, bias_v, sem):
    wid = lax.axis_index("s") * NUM_CORES + lax.axis_index("c")

    pltpu.sync_copy(idx_hbm.at[wid], idx_v)
    pltpu.sync_copy(bias_hbm, bias_v)
    pltpu.async_copy(w_hbm.at[0].at[idx_v], rows_v, sem).wait()

    acc = rows_v[pl.ds(0, BPW)] + bias_v[...]
    for f in range(1, FIELD):
        acc = acc + rows_v[pl.ds(f * BPW, BPW)]
    out_v[...] = acc
    pltpu.sync_copy(out_v, out_hbm.at[pl.ds(wid * BPW, BPW)])


@jax.jit
def _lr_call(idx, w, bias16):
    w_t = w.T
    f = functools.partial(
        pl.kernel,
        mesh=plsc.VectorSubcoreMesh(core_axis_name="c", subcore_axis_name="s"),
        out_type=jax.ShapeDtypeStruct((BATCH,), jnp.float32),
        scratch_types=[
            pltpu.VMEM((IPW,), jnp.int32),
            pltpu.VMEM((IPW,), jnp.float32),
            pltpu.VMEM((BPW,), jnp.float32),
            pltpu.VMEM((BPW,), jnp.float32),
            pltpu.SemaphoreType.DMA,
        ],
    )(_lr_body)
    return f(idx, w_t, bias16).reshape(BATCH, 1)


def kernel(x, W, bias):
    idx = (
        x.astype(jnp.int32)
        .reshape(NW, BPW, FIELD)
        .transpose(0, 2, 1)
        .reshape(NW, IPW)
    )
    bias16 = jnp.broadcast_to(bias.astype(jnp.float32), (BPW,))
    return _lr_call(idx, W, bias16)


# speedup vs baseline: 1.0729x; 1.0729x over previous
import functools

import jax
import jax.numpy as jnp
from jax import lax
from jax.experimental import pallas as pl
from jax.experimental.pallas import tpu as pltpu
from jax.experimental.pallas import tpu_sc as plsc

BATCH = 16384
FIELD = 26
LANES = 16
NUM_CORES = 2
NUM_SUBCORES = 16
NW = NUM_CORES * NUM_SUBCORES
BPW = BATCH // NW
IPW = BPW * FIELD
FEATS = 1000000


def _lr_body(idx_hbm, w_hbm, bias_hbm, out_hbm, idx_v, rows_v, out_v, bias_v, sem):
    wid = lax.axis_index("s") * NUM_CORES + lax.axis_index("c")

    pltpu.sync_copy(idx_hbm.at[wid], idx_v)
    pltpu.sync_copy(bias_hbm, bias_v)
    pltpu.async_copy(w_hbm.at[0].at[idx_v], rows_v, sem).wait()

    bvec = bias_v[...]

    def chunk(j, _):
        base = j * LANES
        acc = rows_v[pl.ds(base, LANES)] + bvec
        for f in range(1, FIELD):
            acc = acc + rows_v[pl.ds(f * BPW + base, LANES)]
        out_v[pl.ds(base, LANES)] = acc
        return 0

    lax.fori_loop(0, BPW // LANES, chunk, 0)
    pltpu.sync_copy(out_v, out_hbm.at[pl.ds(wid * BPW, BPW)])


@jax.jit
def _lr_call(idx, w, bias16):
    w_t = w.T
    f = functools.partial(
        pl.kernel,
        mesh=plsc.VectorSubcoreMesh(core_axis_name="c", subcore_axis_name="s"),
        out_type=jax.ShapeDtypeStruct((BATCH,), jnp.float32),
        scratch_types=[
            pltpu.VMEM((IPW,), jnp.int32),
            pltpu.VMEM((IPW,), jnp.float32),
            pltpu.VMEM((BPW,), jnp.float32),
            pltpu.VMEM((LANES,), jnp.float32),
            pltpu.SemaphoreType.DMA,
        ],
    )(_lr_body)
    return f(idx, w_t, bias16).reshape(BATCH, 1)


def kernel(x, W, bias):
    idx = (
        x.astype(jnp.int32)
        .reshape(NW, BPW, FIELD)
        .transpose(0, 2, 1)
        .reshape(NW, IPW)
    )
    bias16 = jnp.broadcast_to(bias.astype(jnp.float32), (LANES,))
    return _lr_call(idx, W, bias16)
